# baseline (speedup 1.0000x reference)
import dataclasses
import functools

import jax
import jax.numpy as jnp
from jax import lax
from jax.experimental import pallas as pl
from jax.experimental.pallas import tpu as pltpu
from jax.experimental.pallas import tpu_sc as plsc

_K = 8
_NTILES = 32
_LANES = 16
_TILE_ROWS = 256


def _gate_block(x_ref, w1_ref, b1_ref, w2_ref, b2_ref, pt_ref):
    bm, C, DC = x_ref.shape
    h = None
    for c in range(C):
        xc = x_ref[:, c, :]
        w1c = w1_ref[:, pl.ds(c * DC, DC)]
        part = lax.dot_general(
            xc, w1c, (((1,), (1,)), ((), ())),
            preferred_element_type=jnp.float32)
        h = part if h is None else h + part
    h = jnp.maximum(h + b1_ref[...], 0.0)
    nt = pt_ref.shape[0]
    tr = pt_ref.shape[2]
    for s in range(nt):
        hs = h[s * tr:(s + 1) * tr, :]
        lt = lax.dot_general(
            w2_ref[...], hs, (((1,), (1,)), ((), ())),
            preferred_element_type=jnp.float32)
        lt = lt + b2_ref[...]
        m = jnp.max(lt, axis=0, keepdims=True)
        e = jnp.exp(lt - m)
        pt_ref[s] = e / jnp.sum(e, axis=0, keepdims=True)


def _probs_t(tokens, W1, b1, W2, b2, row_start, nrows):
    B, C, DC = tokens.shape
    H = W1.shape[0]
    E = W2.shape[0]
    BM = 512
    blk0 = row_start // BM
    return pl.pallas_call(
        _gate_block,
        grid=(nrows // BM,),
        in_specs=[
            pl.BlockSpec((BM, C, DC), lambda i: (i + blk0, 0, 0)),
            pl.BlockSpec((H, C * DC), lambda i: (0, 0)),
            pl.BlockSpec((1, H), lambda i: (0, 0)),
            pl.BlockSpec((E, H), lambda i: (0, 0)),
            pl.BlockSpec((E, 1), lambda i: (0, 0)),
        ],
        out_specs=pl.BlockSpec(
            (BM // _TILE_ROWS, E, _TILE_ROWS), lambda i: (i, 0, 0)),
        out_shape=jax.ShapeDtypeStruct(
            (nrows // _TILE_ROWS, E, _TILE_ROWS), jnp.float32),
        compiler_params=pltpu.CompilerParams(
            dimension_semantics=("parallel",),
        ),
    )(tokens, W1, b1.reshape(1, H), W2, b2.reshape(E, 1))


def _topk_scatter(probs_t):
    ntiles, E, rows_per_tile = probs_t.shape
    B = ntiles * rows_per_tile
    groups = rows_per_tile // _LANES
    mesh = plsc.VectorSubcoreMesh(core_axis_name="c", subcore_axis_name="s")

    cp = pltpu.CompilerParams()
    if "needs_layout_passes" in pltpu.CompilerParams.__dataclass_fields__:
        cp = dataclasses.replace(cp, needs_layout_passes=False)
    if "use_tc_tiling_on_sc" in pltpu.CompilerParams.__dataclass_fields__:
        cp = dataclasses.replace(cp, use_tc_tiling_on_sc=True)

    @functools.partial(
        pl.kernel,
        compiler_params=cp,
        out_type=[
            jax.ShapeDtypeStruct((B, E), jnp.float32),
            jax.ShapeDtypeStruct((B, _K), jnp.int32),
        ],
        mesh=mesh,
        scratch_types=[
            pltpu.VMEM((E, rows_per_tile), jnp.float32),
            pltpu.VMEM((rows_per_tile, E), jnp.float32),
            pltpu.VMEM((rows_per_tile, _K), jnp.int32),
        ],
    )
    def sc_kernel(pt_hbm, r_hbm, idx_hbm, pt_v, r_v, idx_v):
        wid = lax.axis_index("s") * 2 + lax.axis_index("c")
        base = wid * rows_per_tile
        pltpu.sync_copy(pt_hbm.at[wid], pt_v)

        lane = lax.iota(jnp.int32, _LANES)
        zero_v = jnp.zeros((_LANES,), jnp.float32)

        @pl.loop(0, rows_per_tile)
        def _(r):
            @pl.loop(0, E, step=_LANES)
            def _(c):
                r_v[r, pl.ds(c, _LANES)] = zero_v

        @pl.loop(0, groups)
        def _(g):
            row_vec = g * _LANES + lane
            neg = jnp.full((_LANES,), -1.0, jnp.float32)
            zi = jnp.zeros((_LANES,), jnp.int32)
            carry0 = (neg,) * _K + (zi,) * _K

            def body(e, carry):
                t = list(carry[:_K])
                j = list(carry[_K:])
                cur_v = pt_v[e, pl.ds(g * _LANES, _LANES)]
                cur_j = jnp.full((_LANES,), e, jnp.int32)
                for k in range(_K):
                    gt = cur_v > t[k]
                    nt = jnp.where(gt, cur_v, t[k])
                    nj = jnp.where(gt, cur_j, j[k])
                    cur_v = jnp.where(gt, t[k], cur_v)
                    cur_j = jnp.where(gt, j[k], cur_j)
                    t[k] = nt
                    j[k] = nj
                return tuple(t) + tuple(j)

            carry = lax.fori_loop(0, E, body, carry0)
            t = carry[:_K]
            j = carry[_K:]
            for k in range(_K):
                plsc.store_scatter(r_v, [row_vec, j[k]], t[k])
                plsc.store_scatter(
                    idx_v, [row_vec, jnp.full((_LANES,), k, jnp.int32)], j[k])

        pltpu.sync_copy(r_v, r_hbm.at[pl.ds(base, rows_per_tile), :])
        pltpu.sync_copy(idx_v, idx_hbm.at[pl.ds(base, rows_per_tile), :])

    return sc_kernel(probs_t)


def kernel(tokens, W1, b1, W2, b2):
    B = tokens.shape[0]
    pt = _probs_t(tokens, W1, b1, W2, b2, 0, B)
    R, idx = _topk_scatter(pt)
    return (R, idx)


# speedup vs baseline: 1.0041x; 1.0041x over previous
import dataclasses
import functools

import jax
import jax.numpy as jnp
from jax import lax
from jax.experimental import pallas as pl
from jax.experimental.pallas import tpu as pltpu
from jax.experimental.pallas import tpu_sc as plsc

_K = 8
_NTILES = 32
_LANES = 16
_TILE_ROWS = 256


def _gate_block(x_ref, w1_ref, b1_ref, w2_ref, b2_ref, pt_ref):
    bm, C, DC = x_ref.shape
    h = None
    for c in range(C):
        xc = x_ref[:, c, :]
        w1c = w1_ref[:, pl.ds(c * DC, DC)]
        part = lax.dot_general(
            xc, w1c, (((1,), (1,)), ((), ())),
            preferred_element_type=jnp.float32)
        h = part if h is None else h + part
    h = jnp.maximum(h + b1_ref[...], 0.0)
    nt = pt_ref.shape[0]
    tr = pt_ref.shape[2]
    for s in range(nt):
        hs = h[s * tr:(s + 1) * tr, :]
        lt = lax.dot_general(
            w2_ref[...], hs, (((1,), (1,)), ((), ())),
            preferred_element_type=jnp.float32)
        lt = lt + b2_ref[...]
        m = jnp.max(lt, axis=0, keepdims=True)
        e = jnp.exp(lt - m)
        pt_ref[s] = e / jnp.sum(e, axis=0, keepdims=True)


def _probs_t(tokens, W1, b1, W2, b2, row_start, nrows):
    B, C, DC = tokens.shape
    H = W1.shape[0]
    E = W2.shape[0]
    BM = 512
    blk0 = row_start // BM
    return pl.pallas_call(
        _gate_block,
        grid=(nrows // BM,),
        in_specs=[
            pl.BlockSpec((BM, C, DC), lambda i: (i + blk0, 0, 0)),
            pl.BlockSpec((H, C * DC), lambda i: (0, 0)),
            pl.BlockSpec((1, H), lambda i: (0, 0)),
            pl.BlockSpec((E, H), lambda i: (0, 0)),
            pl.BlockSpec((E, 1), lambda i: (0, 0)),
        ],
        out_specs=pl.BlockSpec(
            (BM // _TILE_ROWS, E, _TILE_ROWS), lambda i: (i, 0, 0)),
        out_shape=jax.ShapeDtypeStruct(
            (nrows // _TILE_ROWS, E, _TILE_ROWS), jnp.float32),
        compiler_params=pltpu.CompilerParams(
            dimension_semantics=("arbitrary",),
        ),
    )(tokens, W1, b1.reshape(1, H), W2, b2.reshape(E, 1))


def _topk_scatter(probs_t):
    ntiles, E, rows_per_tile = probs_t.shape
    B = ntiles * rows_per_tile
    groups = rows_per_tile // _LANES
    mesh = plsc.VectorSubcoreMesh(core_axis_name="c", subcore_axis_name="s")

    cp = pltpu.CompilerParams()
    if "needs_layout_passes" in pltpu.CompilerParams.__dataclass_fields__:
        cp = dataclasses.replace(cp, needs_layout_passes=False)
    if "use_tc_tiling_on_sc" in pltpu.CompilerParams.__dataclass_fields__:
        cp = dataclasses.replace(cp, use_tc_tiling_on_sc=True)

    @functools.partial(
        pl.kernel,
        compiler_params=cp,
        out_type=[
            jax.ShapeDtypeStruct((B, E), jnp.float32),
            jax.ShapeDtypeStruct((B, _K), jnp.int32),
        ],
        mesh=mesh,
        scratch_types=[
            pltpu.VMEM((E, rows_per_tile), jnp.float32),
            pltpu.VMEM((rows_per_tile, E), jnp.float32),
            pltpu.VMEM((rows_per_tile, _K), jnp.int32),
        ],
    )
    def sc_kernel(pt_hbm, r_hbm, idx_hbm, pt_v, r_v, idx_v):
        wid = lax.axis_index("s") * 2 + lax.axis_index("c")
        base = wid * rows_per_tile
        pltpu.sync_copy(pt_hbm.at[wid], pt_v)

        lane = lax.iota(jnp.int32, _LANES)
        zero_v = jnp.zeros((_LANES,), jnp.float32)

        @pl.loop(0, rows_per_tile)
        def _(r):
            @pl.loop(0, E, step=_LANES)
            def _(c):
                r_v[r, pl.ds(c, _LANES)] = zero_v

        @pl.loop(0, groups)
        def _(g):
            row_vec = g * _LANES + lane
            neg = jnp.full((_LANES,), -1.0, jnp.float32)
            zi = jnp.zeros((_LANES,), jnp.int32)
            carry0 = (neg,) * _K + (zi,) * _K

            def body(e2, carry):
                t = list(carry[:_K])
                j = list(carry[_K:])
                for u in range(2):
                    e = e2 * 2 + u
                    cur_v = pt_v[e, pl.ds(g * _LANES, _LANES)]
                    cur_j = jnp.full((_LANES,), e, jnp.int32)
                    for k in range(_K):
                        gt = cur_v > t[k]
                        nt = jnp.where(gt, cur_v, t[k])
                        nj = jnp.where(gt, cur_j, j[k])
                        cur_v = jnp.where(gt, t[k], cur_v)
                        cur_j = jnp.where(gt, j[k], cur_j)
                        t[k] = nt
                        j[k] = nj
                return tuple(t) + tuple(j)

            carry = lax.fori_loop(0, E // 2, body, carry0)
            t = carry[:_K]
            j = carry[_K:]
            for k in range(_K):
                plsc.store_scatter(r_v, [row_vec, j[k]], t[k])
                plsc.store_scatter(
                    idx_v, [row_vec, jnp.full((_LANES,), k, jnp.int32)], j[k])

        pltpu.sync_copy(r_v, r_hbm.at[pl.ds(base, rows_per_tile), :])
        pltpu.sync_copy(idx_v, idx_hbm.at[pl.ds(base, rows_per_tile), :])

    return sc_kernel(probs_t)


def kernel(tokens, W1, b1, W2, b2):
    B = tokens.shape[0]
    pt = _probs_t(tokens, W1, b1, W2, b2, 0, B)
    R, idx = _topk_scatter(pt)
    return (R, idx)
